# baseline (speedup 1.0000x reference)
import functools

import jax
import jax.numpy as jnp
from jax import lax
from jax.experimental import pallas as pl
from jax.experimental.pallas import tpu as pltpu
from jax.experimental.pallas import tpu_sc as plsc

NC = 2
NS = 16
NW = NC * NS
L = 16


def _gather_grid(b_per_w, n_chunks, chunk, D):
    mesh = plsc.VectorSubcoreMesh(core_axis_name="c", subcore_axis_name="s")

    @functools.partial(
        pl.kernel,
        mesh=mesh,
        out_type=jax.ShapeDtypeStruct((NW * b_per_w, D), jnp.float32),
        scratch_types=[
            pltpu.VMEM((b_per_w,), jnp.int32),
            pltpu.VMEM((chunk, D), jnp.float32),
            pltpu.VMEM((chunk, D), jnp.float32),
            pltpu.SemaphoreType.DMA,
            pltpu.SemaphoreType.DMA,
            pltpu.SemaphoreType.DMA,
            pltpu.SemaphoreType.DMA,
            pltpu.SemaphoreType.DMA,
        ],
    )
    def k(atom_hbm, idx_hbm, out_hbm,
          idx_s, r0, r1, cs, gs0, gs1, os0, os1):
        rows_v = (r0, r1)
        gsem = (gs0, gs1)
        osem = (os0, os1)

        wid = lax.axis_index("s") * NC + lax.axis_index("c")
        base_w = wid * b_per_w

        def out_slice(j):
            return out_hbm.at[pl.ds(base_w + j * chunk, chunk), :]

        def out_start(j):
            pltpu.async_copy(rows_v[j % 2], out_slice(j), osem[j % 2])

        def out_wait(j):
            pltpu.make_async_copy(rows_v[j % 2], out_slice(j), osem[j % 2]).wait()

        def gather_start(j):
            pltpu.async_copy(
                atom_hbm.at[idx_s.at[pl.ds(j * chunk, chunk)]],
                rows_v[j % 2], gsem[j % 2])

        def gather_wait(j):
            pltpu.make_async_copy(
                atom_hbm.at[idx_s.at[pl.ds(j * chunk, chunk)]],
                rows_v[j % 2], gsem[j % 2]).wait()

        idx_src = idx_hbm.at[pl.ds(base_w, b_per_w)]
        pltpu.async_copy(idx_src, idx_s, cs)
        pltpu.make_async_copy(idx_src, idx_s, cs).wait()

        for j in range(n_chunks):
            if j >= 2:
                out_wait(j - 2)
            gather_start(j)
            if j >= 1:
                gather_wait(j - 1)
                out_start(j - 1)

        gather_wait(n_chunks - 1)
        out_start(n_chunks - 1)
        if n_chunks >= 2:
            out_wait(n_chunks - 2)
        out_wait(n_chunks - 1)

    return k


def kernel(atom_matrix, connectivity):
    V, D = atom_matrix.shape
    B = connectivity.shape[0]
    assert B % NW == 0
    b_per_w = B // NW
    chunk = 400
    assert b_per_w % chunk == 0 and chunk % L == 0
    n_chunks = b_per_w // chunk
    idx = connectivity[:, 1].astype(jnp.int32)
    return _gather_grid(b_per_w, n_chunks, chunk, D)(atom_matrix, idx)
